# Initial kernel scaffold; baseline (speedup 1.0000x reference)
import jax
import jax.numpy as jnp
from jax.experimental import pallas as pl


def kernel(h, edge_index, W1, b1, W2, b2):
    raise NotImplementedError("write your pallas kernel here")


# speedup vs baseline: 2.2158x; 2.2158x over previous
import functools

import jax
import jax.numpy as jnp
from jax import lax
from jax.experimental import pallas as pl
from jax.experimental.pallas import tpu as pltpu
from jax.experimental.pallas import tpu_sc as plsc

N = 10000
E = 320000
D = 128
H = 256
O = 128

EDGE_BLOCK = 2000
NBLK = E // EDGE_BLOCK
NBANKS = 8
GATHER_WINDOW = 256
NEG = -3.0e38


def _sc_gather(h, gidx):
  mesh = plsc.VectorSubcoreMesh(core_axis_name="core", subcore_axis_name="subcore")

  @functools.partial(
      pl.kernel,
      out_type=jax.ShapeDtypeStruct((2 * E, D), jnp.float32),
      mesh=mesh,
  )
  def gather_kernel(h_hbm, i_hbm, o_hbm):
    def body(i_vmem, o_vmem):
      pltpu.sync_copy(h_hbm.at[i_vmem.at[0]], o_vmem)

    pltpu.emit_pipeline(
        body,
        grid=(2 * E // GATHER_WINDOW,),
        in_specs=[pl.BlockSpec((1, GATHER_WINDOW), index_map=lambda i: (0, i))],
        out_specs=[pl.BlockSpec((GATHER_WINDOW, D), index_map=lambda i: (i, 0))],
        core_axis_name=("core", "subcore"),
        dimension_semantics=(pltpu.PARALLEL,),
    )(i_hbm, o_hbm)

  return gather_kernel(h, gidx)


def _elu(x):
  return jnp.where(x > 0, x, jnp.exp(jnp.minimum(x, 0.0)) - 1.0)


def _edge_kernel(gd_ref, gs_ref, w1at_ref, w1bt_ref, w2t_ref, b1_ref, b2_ref,
                 dst_ref, out_ref, acc_ref, m2_ref):
  i = pl.program_id(0)

  @pl.when(i == 0)
  def _init():
    acc_ref[...] = jnp.full(acc_ref.shape, NEG, jnp.float32)

  pre1 = (
      jnp.dot(gd_ref[...], w1at_ref[...], preferred_element_type=jnp.float32)
      + jnp.dot(gs_ref[...], w1bt_ref[...], preferred_element_type=jnp.float32)
      + b1_ref[...]
  )
  m1 = _elu(pre1)
  z = jnp.dot(m1, w2t_ref[...], preferred_element_type=jnp.float32) + b2_ref[...]
  m2_ref[...] = z

  def body(j, carry):
    base = j * NBANKS
    for k in range(NBANKS):
      e = base + k
      idx = dst_ref[0, 0, e]
      row = m2_ref[pl.ds(e, 1), :]
      bank = acc_ref.at[k]
      cur = bank[pl.ds(idx, 1), :]
      bank[pl.ds(idx, 1), :] = jnp.maximum(cur, row)
    return carry

  lax.fori_loop(0, EDGE_BLOCK // NBANKS, body, 0)

  @pl.when(i == NBLK - 1)
  def _finalize():
    m = acc_ref[0]
    for k in range(1, NBANKS):
      m = jnp.maximum(m, acc_ref[k])
    out_ref[...] = jnp.where(m < -1.0e38, 0.0, _elu(m))


def kernel(h, edge_index, W1, b1, W2, b2):
  src = edge_index[0]
  dst = edge_index[1]
  gidx = jnp.concatenate([dst, src]).reshape(1, 2 * E)
  g = _sc_gather(h, gidx)

  w1at = W1[:, :D].T
  w1bt = W1[:, D:].T
  w2t = W2.T
  dstb = dst.reshape(NBLK, 1, EDGE_BLOCK)

  out = pl.pallas_call(
      _edge_kernel,
      grid=(NBLK,),
      in_specs=[
          pl.BlockSpec((EDGE_BLOCK, D), lambda i: (i, 0)),
          pl.BlockSpec((EDGE_BLOCK, D), lambda i: (i + NBLK, 0)),
          pl.BlockSpec((D, H), lambda i: (0, 0)),
          pl.BlockSpec((D, H), lambda i: (0, 0)),
          pl.BlockSpec((H, O), lambda i: (0, 0)),
          pl.BlockSpec((1, H), lambda i: (0, 0)),
          pl.BlockSpec((1, O), lambda i: (0, 0)),
          pl.BlockSpec((1, 1, EDGE_BLOCK), lambda i: (i, 0, 0),
                       memory_space=pltpu.MemorySpace.SMEM),
      ],
      out_specs=pl.BlockSpec((N, O), lambda i: (0, 0)),
      out_shape=jax.ShapeDtypeStruct((N, O), jnp.float32),
      scratch_shapes=[
          pltpu.VMEM((NBANKS, N, O), jnp.float32),
          pltpu.VMEM((EDGE_BLOCK, O), jnp.float32),
      ],
      compiler_params=pltpu.CompilerParams(
          dimension_semantics=("arbitrary",),
          vmem_limit_bytes=100 * 1024 * 1024,
      ),
  )(g, g, w1at, w1bt, w2t, b1.reshape(1, H), b2.reshape(1, O), dstb)
  return out


# speedup vs baseline: 2.2162x; 1.0002x over previous
import functools

import jax
import jax.numpy as jnp
from jax import lax
from jax.experimental import pallas as pl
from jax.experimental.pallas import tpu as pltpu
from jax.experimental.pallas import tpu_sc as plsc

N = 10000
E = 320000
D = 128
H = 256
O = 128

EDGE_BLOCK = 2000
NBLK = E // EDGE_BLOCK
NBANKS = 8
GATHER_WINDOW = 256
NEG = -3.0e38


def _sc_gather(h, gidx):
  mesh = plsc.VectorSubcoreMesh(core_axis_name="core", subcore_axis_name="subcore")

  @functools.partial(
      pl.kernel,
      out_type=jax.ShapeDtypeStruct((2 * E, D), jnp.float32),
      mesh=mesh,
  )
  def gather_kernel(h_hbm, i_hbm, o_hbm):
    def body(i_vmem, o_vmem):
      pltpu.sync_copy(h_hbm.at[i_vmem.at[0]], o_vmem)

    pltpu.emit_pipeline(
        body,
        grid=(2 * E // GATHER_WINDOW,),
        in_specs=[pl.BlockSpec((1, GATHER_WINDOW), index_map=lambda i: (0, i))],
        out_specs=[pl.BlockSpec((GATHER_WINDOW, D), index_map=lambda i: (i, 0))],
        core_axis_name=("core", "subcore"),
        dimension_semantics=(pltpu.PARALLEL,),
    )(i_hbm, o_hbm)

  return gather_kernel(h, gidx)


def _elu(x):
  return jnp.where(x > 0, x, jnp.exp(jnp.minimum(x, 0.0)) - 1.0)


def _edge_kernel(gd_ref, gs_ref, w1at_ref, w1bt_ref, w2t_ref, b1_ref, b2_ref,
                 dst_ref, out_ref, acc_ref, m2_ref):
  i = pl.program_id(0)

  @pl.when(i == 0)
  def _init():
    acc_ref[...] = jnp.full(acc_ref.shape, NEG, jnp.float32)

  pre1 = (
      jnp.dot(gd_ref[...], w1at_ref[...], preferred_element_type=jnp.float32)
      + jnp.dot(gs_ref[...], w1bt_ref[...], preferred_element_type=jnp.float32)
      + b1_ref[...]
  )
  m1 = _elu(pre1)
  z = jnp.dot(m1, w2t_ref[...], preferred_element_type=jnp.float32) + b2_ref[...]
  m2_ref[...] = z

  def body(j, carry):
    base = pl.multiple_of(j * NBANKS, NBANKS)
    chunk = m2_ref[pl.ds(base, NBANKS), :]
    for k in range(NBANKS):
      idx = dst_ref[0, 0, j * NBANKS + k]
      row = chunk[k:k + 1, :]
      bank = acc_ref.at[k]
      cur = bank[pl.ds(idx, 1), :]
      bank[pl.ds(idx, 1), :] = jnp.maximum(cur, row)
    return carry

  lax.fori_loop(0, EDGE_BLOCK // NBANKS, body, 0)

  @pl.when(i == NBLK - 1)
  def _finalize():
    m = acc_ref[0]
    for k in range(1, NBANKS):
      m = jnp.maximum(m, acc_ref[k])
    out_ref[...] = jnp.where(m < -1.0e38, 0.0, _elu(m))


def kernel(h, edge_index, W1, b1, W2, b2):
  src = edge_index[0]
  dst = edge_index[1]
  gidx = jnp.concatenate([dst, src]).reshape(1, 2 * E)
  g = _sc_gather(h, gidx)

  w1at = W1[:, :D].T
  w1bt = W1[:, D:].T
  w2t = W2.T
  dstb = dst.reshape(NBLK, 1, EDGE_BLOCK)

  out = pl.pallas_call(
      _edge_kernel,
      grid=(NBLK,),
      in_specs=[
          pl.BlockSpec((EDGE_BLOCK, D), lambda i: (i, 0)),
          pl.BlockSpec((EDGE_BLOCK, D), lambda i: (i + NBLK, 0)),
          pl.BlockSpec((D, H), lambda i: (0, 0)),
          pl.BlockSpec((D, H), lambda i: (0, 0)),
          pl.BlockSpec((H, O), lambda i: (0, 0)),
          pl.BlockSpec((1, H), lambda i: (0, 0)),
          pl.BlockSpec((1, O), lambda i: (0, 0)),
          pl.BlockSpec((1, 1, EDGE_BLOCK), lambda i: (i, 0, 0),
                       memory_space=pltpu.MemorySpace.SMEM),
      ],
      out_specs=pl.BlockSpec((N, O), lambda i: (0, 0)),
      out_shape=jax.ShapeDtypeStruct((N, O), jnp.float32),
      scratch_shapes=[
          pltpu.VMEM((NBANKS, N, O), jnp.float32),
          pltpu.VMEM((EDGE_BLOCK, O), jnp.float32),
      ],
      compiler_params=pltpu.CompilerParams(
          dimension_semantics=("arbitrary",),
          vmem_limit_bytes=100 * 1024 * 1024,
      ),
  )(g, g, w1at, w1bt, w2t, b1.reshape(1, H), b2.reshape(1, O), dstb)
  return out


# speedup vs baseline: 3.2017x; 1.4447x over previous
import functools

import jax
import jax.numpy as jnp
from jax import lax
from jax.experimental import pallas as pl
from jax.experimental.pallas import tpu as pltpu
from jax.experimental.pallas import tpu_sc as plsc

N = 10000
E = 320000
D = 128
H = 256
O = 128

EDGE_BLOCK = 2000
NBLK = E // EDGE_BLOCK
NBANKS = 8
GATHER_WINDOW = 256
NEG = -3.0e38


def _sc_gather(h, gidx):
  mesh = plsc.VectorSubcoreMesh(core_axis_name="core", subcore_axis_name="subcore")

  @functools.partial(
      pl.kernel,
      out_type=jax.ShapeDtypeStruct((2 * E, D), jnp.float32),
      mesh=mesh,
  )
  def gather_kernel(h_hbm, i_hbm, o_hbm):
    def body(i_vmem, o_vmem):
      pltpu.sync_copy(h_hbm.at[i_vmem.at[0]], o_vmem)

    pltpu.emit_pipeline(
        body,
        grid=(2 * E // GATHER_WINDOW,),
        in_specs=[pl.BlockSpec((1, GATHER_WINDOW), index_map=lambda i: (0, i))],
        out_specs=[pl.BlockSpec((GATHER_WINDOW, D), index_map=lambda i: (i, 0))],
        core_axis_name=("core", "subcore"),
        dimension_semantics=(pltpu.PARALLEL,),
    )(i_hbm, o_hbm)

  return gather_kernel(h, gidx)


def _elu(x):
  return jnp.where(x > 0, x, jnp.exp(jnp.minimum(x, 0.0)) - 1.0)


def _edge_kernel(gd_ref, gs_ref, w1at_ref, w1bt_ref, w2t_ref, b1_ref, b2_ref,
                 dst_ref, out_ref, *scratch):
  banks = scratch[:NBANKS]
  m2_ref = scratch[NBANKS]
  i = pl.program_id(0)

  @pl.when(i == 0)
  def _init():
    for b in banks:
      b[...] = jnp.full(b.shape, NEG, jnp.float32)

  pre1 = (
      jnp.dot(gd_ref[...], w1at_ref[...], preferred_element_type=jnp.float32)
      + jnp.dot(gs_ref[...], w1bt_ref[...], preferred_element_type=jnp.float32)
      + b1_ref[...]
  )
  m1 = _elu(pre1)
  z = jnp.dot(m1, w2t_ref[...], preferred_element_type=jnp.float32) + b2_ref[...]
  m2_ref[...] = z

  def body(j, carry):
    base = pl.multiple_of(j * NBANKS, NBANKS)
    chunk = m2_ref[pl.ds(base, NBANKS), :]
    for k in range(NBANKS):
      idx = dst_ref[0, 0, j * NBANKS + k]
      row = chunk[k:k + 1, :]
      bank = banks[k]
      cur = bank[pl.ds(idx, 1), :]
      bank[pl.ds(idx, 1), :] = jnp.maximum(cur, row)
    return carry

  lax.fori_loop(0, EDGE_BLOCK // NBANKS, body, 0)

  @pl.when(i == NBLK - 1)
  def _finalize():
    m = banks[0][...]
    for k in range(1, NBANKS):
      m = jnp.maximum(m, banks[k][...])
    out_ref[...] = jnp.where(m < -1.0e38, 0.0, _elu(m))


def kernel(h, edge_index, W1, b1, W2, b2):
  src = edge_index[0]
  dst = edge_index[1]
  gidx = jnp.concatenate([dst, src]).reshape(1, 2 * E)
  g = _sc_gather(h, gidx)

  w1at = W1[:, :D].T
  w1bt = W1[:, D:].T
  w2t = W2.T
  dstb = dst.reshape(NBLK, 1, EDGE_BLOCK)

  out = pl.pallas_call(
      _edge_kernel,
      grid=(NBLK,),
      in_specs=[
          pl.BlockSpec((EDGE_BLOCK, D), lambda i: (i, 0)),
          pl.BlockSpec((EDGE_BLOCK, D), lambda i: (i + NBLK, 0)),
          pl.BlockSpec((D, H), lambda i: (0, 0)),
          pl.BlockSpec((D, H), lambda i: (0, 0)),
          pl.BlockSpec((H, O), lambda i: (0, 0)),
          pl.BlockSpec((1, H), lambda i: (0, 0)),
          pl.BlockSpec((1, O), lambda i: (0, 0)),
          pl.BlockSpec((1, 1, EDGE_BLOCK), lambda i: (i, 0, 0),
                       memory_space=pltpu.MemorySpace.SMEM),
      ],
      out_specs=pl.BlockSpec((N, O), lambda i: (0, 0)),
      out_shape=jax.ShapeDtypeStruct((N, O), jnp.float32),
      scratch_shapes=(
          [pltpu.VMEM((N, O), jnp.float32) for _ in range(NBANKS)]
          + [pltpu.VMEM((EDGE_BLOCK, O), jnp.float32)]
      ),
      compiler_params=pltpu.CompilerParams(
          dimension_semantics=("arbitrary",),
          vmem_limit_bytes=100 * 1024 * 1024,
      ),
  )(g, g, w1at, w1bt, w2t, b1.reshape(1, H), b2.reshape(1, O), dstb)
  return out
